# baseline (speedup 1.0000x reference)
import jax
import jax.numpy as jnp
import numpy as np
from jax.experimental import pallas as pl
from jax.experimental.pallas import tpu as pltpu

_B = 64
_EMBED = 768
_NVIS = 256
_NMASK = 768
_NROWS = 1025


def _build_maps(sidx_ref, midx_ref, src_ref, perm_ref):
    def init(i, _):
        src_ref[i] = -1
        return 0

    jax.lax.fori_loop(0, _NROWS, init, 0)
    src_ref[0] = 0

    def samp(j, _):
        src_ref[sidx_ref[j]] = j + 1
        return 0

    jax.lax.fori_loop(0, _NVIS, samp, 0)

    def msk(j, _):
        src_ref[midx_ref[j]] = -2
        return 0

    jax.lax.fori_loop(0, _NMASK, msk, 0)

    def emit(lo, hi, c0):
        def body(r, c):
            s = src_ref[r]
            hit = jnp.logical_and(s >= lo, s <= hi)

            @pl.when(hit)
            def _():
                perm_ref[c] = r

            return c + jnp.where(hit, 1, 0)

        return jax.lax.fori_loop(0, _NROWS, body, c0)

    c = emit(-2, -2, 0)
    c = emit(-1, -1, c)
    emit(0, _NVIS, c)


def _assemble(src_ref, perm_ref, x_ref, w0_ref, b_ref, o_ref):
    t = pl.program_id(0)
    s_val = jnp.sum(w0_ref[...]) * np.float32(127.0 / 255.0) + b_ref[0, 0]
    src = src_ref[perm_ref[t]]
    xb = x_ref[...]
    o_ref[...] = jnp.where(
        src == -2,
        jnp.full_like(xb, s_val),
        jnp.where(src == -1, jnp.zeros_like(xb), xb),
    )


def kernel(x, sample_index, mask_index, W, b):
    src, perm = pl.pallas_call(
        _build_maps,
        in_specs=[
            pl.BlockSpec(memory_space=pltpu.SMEM),
            pl.BlockSpec(memory_space=pltpu.SMEM),
        ],
        out_specs=[
            pl.BlockSpec(memory_space=pltpu.SMEM),
            pl.BlockSpec(memory_space=pltpu.SMEM),
        ],
        out_shape=[
            jax.ShapeDtypeStruct((_NROWS,), jnp.int32),
            jax.ShapeDtypeStruct((_NROWS,), jnp.int32),
        ],
    )(sample_index, mask_index)

    x4 = jnp.reshape(x, (_B, 1 + _NVIS, 1, _EMBED))
    w0 = jnp.reshape(W[0], (1, _EMBED))
    b2 = jnp.reshape(b, (1, _EMBED))

    grid_spec = pltpu.PrefetchScalarGridSpec(
        num_scalar_prefetch=2,
        grid=(_NROWS,),
        in_specs=[
            pl.BlockSpec(
                (_B, 1, 1, _EMBED),
                lambda t, src_r, perm_r: (0, jnp.maximum(src_r[perm_r[t]], 0), 0, 0),
            ),
            pl.BlockSpec((1, _EMBED), lambda t, src_r, perm_r: (0, 0)),
            pl.BlockSpec((1, _EMBED), lambda t, src_r, perm_r: (0, 0)),
        ],
        out_specs=pl.BlockSpec(
            (_B, 1, 1, _EMBED),
            lambda t, src_r, perm_r: (0, perm_r[t], 0, 0),
        ),
    )

    out = pl.pallas_call(
        _assemble,
        grid_spec=grid_spec,
        out_shape=jax.ShapeDtypeStruct((_B, _NROWS, 1, _EMBED), jnp.float32),
    )(src, perm, x4, w0, b2)

    return jnp.reshape(out, (_B, _NROWS, _EMBED))


# speedup vs baseline: 2.3537x; 2.3537x over previous
import jax
import jax.numpy as jnp
import numpy as np
from jax.experimental import pallas as pl
from jax.experimental.pallas import tpu as pltpu

_B = 64
_EMBED = 768
_NVIS = 256
_NMASK = 768
_NROWS = 1025
_R = 16
_GRID = (_NROWS + _R - 1) // _R
_NPAD = _GRID * _R


def _build_maps(sidx_ref, midx_ref, src_ref):
    def init(i, _):
        src_ref[i] = -1
        return 0

    jax.lax.fori_loop(0, _NPAD, init, 0)
    src_ref[0] = 0

    def samp(j, _):
        src_ref[sidx_ref[j]] = j + 1
        return 0

    jax.lax.fori_loop(0, _NVIS, samp, 0)

    def msk(j, _):
        src_ref[midx_ref[j]] = -2
        return 0

    jax.lax.fori_loop(0, _NMASK, msk, 0)


def _assemble(src_ref, x_hbm, w0_ref, b_ref, o_ref, x_vmem, sem):
    t = pl.program_id(0)

    @pl.when(t == 0)
    def _():
        cp = pltpu.make_async_copy(x_hbm, x_vmem, sem)
        cp.start()
        cp.wait()

    s_val = jnp.sum(w0_ref[...]) * np.float32(127.0 / 255.0) + b_ref[0, 0]

    for i in range(_R):
        src = src_ref[t * _R + i]

        @pl.when(src >= 0)
        def _():
            o_ref[:, pl.ds(i, 1)] = x_vmem[:, pl.ds(src, 1)]

        @pl.when(src == -1)
        def _():
            o_ref[:, pl.ds(i, 1)] = jnp.zeros((_B, 1, 1, _EMBED), jnp.float32)

        @pl.when(src == -2)
        def _():
            o_ref[:, pl.ds(i, 1)] = jnp.full((_B, 1, 1, _EMBED), s_val)


def kernel(x, sample_index, mask_index, W, b):
    src = pl.pallas_call(
        _build_maps,
        in_specs=[
            pl.BlockSpec(memory_space=pltpu.SMEM),
            pl.BlockSpec(memory_space=pltpu.SMEM),
        ],
        out_specs=pl.BlockSpec(memory_space=pltpu.SMEM),
        out_shape=jax.ShapeDtypeStruct((_NPAD,), jnp.int32),
    )(sample_index, mask_index)

    x4 = jnp.reshape(x, (_B, 1 + _NVIS, 1, _EMBED))
    w0 = jnp.reshape(W[0], (1, _EMBED))
    b2 = jnp.reshape(b, (1, _EMBED))

    grid_spec = pltpu.PrefetchScalarGridSpec(
        num_scalar_prefetch=1,
        grid=(_GRID,),
        in_specs=[
            pl.BlockSpec(memory_space=pl.ANY),
            pl.BlockSpec((1, _EMBED), lambda t, src_r: (0, 0)),
            pl.BlockSpec((1, _EMBED), lambda t, src_r: (0, 0)),
        ],
        out_specs=pl.BlockSpec(
            (_B, _R, 1, _EMBED), lambda t, src_r: (0, t, 0, 0)
        ),
        scratch_shapes=[
            pltpu.VMEM((_B, 1 + _NVIS, 1, _EMBED), jnp.float32),
            pltpu.SemaphoreType.DMA,
        ],
    )

    out = pl.pallas_call(
        _assemble,
        grid_spec=grid_spec,
        out_shape=jax.ShapeDtypeStruct((_B, _NROWS, 1, _EMBED), jnp.float32),
    )(src, x4, w0, b2)

    return jnp.reshape(out, (_B, _NROWS, _EMBED))


# speedup vs baseline: 2.4926x; 1.0590x over previous
import jax
import jax.numpy as jnp
import numpy as np
from jax.experimental import pallas as pl
from jax.experimental.pallas import tpu as pltpu

_B = 64
_EMBED = 768
_NVIS = 256
_NMASK = 768
_NROWS = 1025
_R = 128
_GRID = (_NROWS + _R - 1) // _R
_NPAD = _GRID * _R


def _build_maps(sidx_ref, midx_ref, src_ref):
    def init(i, _):
        src_ref[i] = -1
        return 0

    jax.lax.fori_loop(0, _NPAD, init, 0)
    src_ref[0] = 0

    def samp(j, _):
        src_ref[sidx_ref[j]] = j + 1
        return 0

    jax.lax.fori_loop(0, _NVIS, samp, 0)

    def msk(j, _):
        src_ref[midx_ref[j]] = -2
        return 0

    jax.lax.fori_loop(0, _NMASK, msk, 0)


def _assemble(src_ref, x_hbm, w0_ref, b_ref, o_ref, sem, cnt_ref):
    t = pl.program_id(0)
    s_val = jnp.sum(w0_ref[...]) * np.float32(127.0 / 255.0) + b_ref[0, 0]
    cnt_ref[0] = 0

    def row(i, c):
        src = src_ref[t * _R + i]

        @pl.when(src >= 0)
        def _():
            pltpu.make_async_copy(
                x_hbm.at[:, pl.ds(src, 1)], o_ref.at[:, pl.ds(i, 1)], sem
            ).start()
            cnt_ref[0] = cnt_ref[0] + 1

        @pl.when(src == -1)
        def _():
            o_ref[:, pl.ds(i, 1)] = jnp.zeros((_B, 1, 1, _EMBED), jnp.float32)

        @pl.when(src == -2)
        def _():
            o_ref[:, pl.ds(i, 1)] = jnp.full((_B, 1, 1, _EMBED), s_val)

        return c

    jax.lax.fori_loop(0, _R, row, 0)

    def drain(k, c):
        pltpu.make_async_copy(
            x_hbm.at[:, pl.ds(0, 1)], o_ref.at[:, pl.ds(0, 1)], sem
        ).wait()
        return c

    jax.lax.fori_loop(0, cnt_ref[0], drain, 0)


def kernel(x, sample_index, mask_index, W, b):
    src = pl.pallas_call(
        _build_maps,
        in_specs=[
            pl.BlockSpec(memory_space=pltpu.SMEM),
            pl.BlockSpec(memory_space=pltpu.SMEM),
        ],
        out_specs=pl.BlockSpec(memory_space=pltpu.SMEM),
        out_shape=jax.ShapeDtypeStruct((_NPAD,), jnp.int32),
    )(sample_index, mask_index)

    x4 = jnp.reshape(x, (_B, 1 + _NVIS, 1, _EMBED))
    w0 = jnp.reshape(W[0], (1, _EMBED))
    b2 = jnp.reshape(b, (1, _EMBED))

    grid_spec = pltpu.PrefetchScalarGridSpec(
        num_scalar_prefetch=1,
        grid=(_GRID,),
        in_specs=[
            pl.BlockSpec(memory_space=pl.ANY),
            pl.BlockSpec((1, _EMBED), lambda t, src_r: (0, 0)),
            pl.BlockSpec((1, _EMBED), lambda t, src_r: (0, 0)),
        ],
        out_specs=pl.BlockSpec(
            (_B, _R, 1, _EMBED), lambda t, src_r: (0, t, 0, 0)
        ),
        scratch_shapes=[
            pltpu.SemaphoreType.DMA,
            pltpu.SMEM((1,), jnp.int32),
        ],
    )

    out = pl.pallas_call(
        _assemble,
        grid_spec=grid_spec,
        out_shape=jax.ShapeDtypeStruct((_B, _NROWS, 1, _EMBED), jnp.float32),
    )(src, x4, w0, b2)

    return jnp.reshape(out, (_B, _NROWS, _EMBED))
